# baseline (speedup 1.0000x reference)
import jax
import jax.numpy as jnp
from jax.experimental import pallas as pl
from jax.experimental.pallas import tpu as pltpu

_N_TILES = 4
_NUM_TOKENS = 1025
_EMBED_DIM = 1280


def _body(ar_ref, gate_ref, x_ref, local_ref, gpe_hbm, out_ref, scratch_ref, sem):
    i = pl.program_id(0)
    b = i // _N_TILES
    t = i % _N_TILES
    h = ar_ref[b, 0]
    w = ar_ref[b, 1]
    w_safe = jnp.maximum(w, 1)
    row = t // w_safe
    col = t % w_safe
    valid = t < h * w

    tg = jnp.tanh(gate_ref[0])
    fetch = jnp.logical_and(valid, tg != 0.0)

    @pl.when(fetch)
    def _start():
        pltpu.make_async_copy(gpe_hbm.at[row, col], scratch_ref, sem).start()

    out_ref[0] = x_ref[0] + local_ref[...] * (1.0 - tg)

    @pl.when(fetch)
    def _finish():
        pltpu.make_async_copy(gpe_hbm.at[row, col], scratch_ref, sem).wait()
        out_ref[0] += scratch_ref[...] * tg


def kernel(x, aspect_ratio, global_positional_embedding, local_positional_embedding, gate):
    bsz, n_tiles, num_tokens, embed_dim = x.shape
    x3 = x.reshape(bsz * n_tiles, num_tokens, embed_dim)
    ar = aspect_ratio.astype(jnp.int32)

    out = pl.pallas_call(
        _body,
        grid=(bsz * n_tiles,),
        in_specs=[
            pl.BlockSpec(memory_space=pltpu.SMEM),
            pl.BlockSpec(memory_space=pltpu.SMEM),
            pl.BlockSpec((1, num_tokens, embed_dim), lambda i: (i, 0, 0)),
            pl.BlockSpec((num_tokens, embed_dim), lambda i: (0, 0)),
            pl.BlockSpec(memory_space=pltpu.MemorySpace.HBM),
        ],
        out_specs=pl.BlockSpec((1, num_tokens, embed_dim), lambda i: (i, 0, 0)),
        out_shape=jax.ShapeDtypeStruct((bsz * n_tiles, num_tokens, embed_dim), x.dtype),
        scratch_shapes=[
            pltpu.VMEM((num_tokens, embed_dim), jnp.float32),
            pltpu.SemaphoreType.DMA,
        ],
    )(ar, gate, x3, local_positional_embedding, global_positional_embedding)
    return out.reshape(bsz, n_tiles, num_tokens, embed_dim)


# speedup vs baseline: 2.9772x; 2.9772x over previous
import jax
import jax.numpy as jnp
from jax.experimental import pallas as pl
from jax.experimental.pallas import tpu as pltpu

_N_TILES = 4


def _body(ar_ref, gate_ref, x_ref, local_ref, gpe_hbm, out_ref, scratch_ref, sem):
    i = pl.program_id(0)
    b = i // _N_TILES
    t = i % _N_TILES
    h = ar_ref[b, 0]
    w = ar_ref[b, 1]
    w_safe = jnp.maximum(w, 1)
    row = t // w_safe
    col = t % w_safe
    valid = t < h * w

    tg = jnp.tanh(gate_ref[0])
    fetch = jnp.logical_and(valid, tg != 0.0)

    @pl.when(fetch)
    def _start():
        pltpu.make_async_copy(gpe_hbm.at[row, col], scratch_ref, sem).start()

    out_ref[0, 0] = x_ref[0, 0] + local_ref[...] * (1.0 - tg)

    @pl.when(fetch)
    def _finish():
        pltpu.make_async_copy(gpe_hbm.at[row, col], scratch_ref, sem).wait()
        out_ref[0, 0] += scratch_ref[...] * tg


def kernel(x, aspect_ratio, global_positional_embedding, local_positional_embedding, gate):
    bsz, n_tiles, num_tokens, embed_dim = x.shape
    ar = aspect_ratio.astype(jnp.int32)

    return pl.pallas_call(
        _body,
        grid=(bsz * n_tiles,),
        in_specs=[
            pl.BlockSpec(memory_space=pltpu.SMEM),
            pl.BlockSpec(memory_space=pltpu.SMEM),
            pl.BlockSpec((1, 1, num_tokens, embed_dim),
                         lambda i: (i // _N_TILES, i % _N_TILES, 0, 0)),
            pl.BlockSpec((num_tokens, embed_dim), lambda i: (0, 0)),
            pl.BlockSpec(memory_space=pltpu.MemorySpace.HBM),
        ],
        out_specs=pl.BlockSpec((1, 1, num_tokens, embed_dim),
                               lambda i: (i // _N_TILES, i % _N_TILES, 0, 0)),
        out_shape=jax.ShapeDtypeStruct((bsz, n_tiles, num_tokens, embed_dim), x.dtype),
        scratch_shapes=[
            pltpu.VMEM((num_tokens, embed_dim), jnp.float32),
            pltpu.SemaphoreType.DMA,
        ],
    )(ar, gate, x, local_positional_embedding, global_positional_embedding)
